# baseline (speedup 1.0000x reference)
import jax
import jax.numpy as jnp
from jax.experimental import pallas as pl
from jax.experimental.pallas import tpu as pltpu

B = 1024
F = 128
BITS = 8
ENC = F * BITS
H = 4096
C = 128
THR = 3.0
VMIN = 0.0
VMAX = 1.0

HH = H // 2
BH1 = 512
BH2 = 256
N1 = HH // BH1
N2 = HH // BH2


def _fused_kernel(x_ref, w0a_ref, w0b_ref, w1a_ref, w1b_ref, outc_ref,
                  preds_ref, outact_ref, enc_ref, a0_ref, acc_ref):
    i = pl.program_id(0)

    @pl.when(i == 0)
    def _encode():
        xc = jnp.clip(x_ref[...], VMIN, VMAX)
        norm = (xc - VMIN) / (VMAX - VMIN)
        scaled = jnp.round(norm * (2 ** BITS - 1)).astype(jnp.int32)
        gray = scaled ^ (scaled >> 1)
        rowf = jax.lax.broadcasted_iota(jnp.int32, (F, ENC), 0)
        colf = jax.lax.broadcasted_iota(jnp.int32, (F, ENC), 1)
        sel = (colf // BITS == rowf).astype(jnp.float32)
        gexp = jnp.dot(gray.astype(jnp.float32), sel,
                       preferred_element_type=jnp.float32)
        bitpos = jax.lax.broadcasted_iota(jnp.int32, (B, ENC), 1) % BITS
        bits = (gexp.astype(jnp.int32) >> bitpos) & 1
        enc_ref[...] = bits.astype(jnp.bfloat16)
        acc_ref[...] = jnp.zeros((B, C), jnp.float32)

    def _l1_chunk(w_ref, col0):
        w = w_ref[...].astype(jnp.bfloat16)
        z = jax.lax.dot_general(enc_ref[...], w, (((1,), (1,)), ((), ())),
                                preferred_element_type=jnp.float32)
        a0 = (z >= THR).astype(jnp.float32)
        a0_ref[:, pl.ds(col0, BH1)] = a0.astype(jnp.bfloat16)
        c0 = outc_ref[0, pl.ds(col0, BH1), :]
        acc_ref[...] += jnp.dot(a0, c0, preferred_element_type=jnp.float32)

    def _l2_chunk(w_ref, col0):
        w = w_ref[...].astype(jnp.bfloat16)
        z = jax.lax.dot_general(a0_ref[...], w, (((1,), (1,)), ((), ())),
                                preferred_element_type=jnp.float32)
        a1 = (z >= THR).astype(jnp.float32)
        c1 = outc_ref[1, pl.ds(col0, BH2), :]
        acc_ref[...] += jnp.dot(a1, c1, preferred_element_type=jnp.float32)

    @pl.when(i < N1)
    def _layer1():
        _l1_chunk(w0a_ref, i * BH1)
        _l1_chunk(w0b_ref, HH + i * BH1)

    @pl.when(i >= N1)
    def _layer2():
        k = i - N1
        _l2_chunk(w1a_ref, k * BH2)
        _l2_chunk(w1b_ref, HH + k * BH2)

        @pl.when(k == N2 - 1)
        def _finish():
            out = acc_ref[...]
            outact_ref[...] = out
            preds_ref[0, :] = jnp.argmax(out, axis=1).astype(jnp.int32)


def kernel(trainOrTest, x, y, W0, W1, outC):
    preds2, outAct = pl.pallas_call(
        _fused_kernel,
        grid=(N1 + N2,),
        in_specs=[
            pl.BlockSpec((B, F), lambda i: (0, 0)),
            pl.BlockSpec((BH1, ENC), lambda i: (jnp.minimum(i, N1 - 1), 0)),
            pl.BlockSpec((BH1, ENC),
                         lambda i: (HH // BH1 + jnp.minimum(i, N1 - 1), 0)),
            pl.BlockSpec((BH2, H), lambda i: (jnp.maximum(i - N1, 0), 0)),
            pl.BlockSpec((BH2, H),
                         lambda i: (HH // BH2 + jnp.maximum(i - N1, 0), 0)),
            pl.BlockSpec((2, H, C), lambda i: (0, 0, 0)),
        ],
        out_specs=[
            pl.BlockSpec((1, B), lambda i: (0, 0)),
            pl.BlockSpec((B, C), lambda i: (0, 0)),
        ],
        out_shape=[
            jax.ShapeDtypeStruct((1, B), jnp.int32),
            jax.ShapeDtypeStruct((B, C), jnp.float32),
        ],
        scratch_shapes=[
            pltpu.VMEM((B, ENC), jnp.bfloat16),
            pltpu.VMEM((B, H), jnp.bfloat16),
            pltpu.VMEM((B, C), jnp.float32),
        ],
        compiler_params=pltpu.CompilerParams(
            dimension_semantics=("arbitrary",),
        ),
    )(x, W0, W0, W1, W1, outC)
    return preds2[0], outAct


# speedup vs baseline: 1.6013x; 1.6013x over previous
import jax
import jax.numpy as jnp
from jax.experimental import pallas as pl
from jax.experimental.pallas import tpu as pltpu

B = 1024
F = 128
BITS = 8
ENC = F * BITS
H = 4096
C = 128
THR = 3.0
VMIN = 0.0
VMAX = 1.0

BH1 = 512
BH2 = 512
N1 = H // BH1
N2 = H // BH2


def _fused_kernel(x_ref, w0_ref, w1_ref, outc_ref, preds_ref, outact_ref,
                  enc_ref, a0_ref, acc_ref):
    i = pl.program_id(0)

    @pl.when(i == 0)
    def _encode():
        xc = jnp.clip(x_ref[...], VMIN, VMAX)
        norm = (xc - VMIN) / (VMAX - VMIN)
        scaled = jnp.round(norm * (2 ** BITS - 1)).astype(jnp.int32)
        gray = scaled ^ (scaled >> 1)
        rowf = jax.lax.broadcasted_iota(jnp.int32, (F, ENC), 0)
        colf = jax.lax.broadcasted_iota(jnp.int32, (F, ENC), 1)
        sel = (colf // BITS == rowf).astype(jnp.float32)
        gexp = jnp.dot(gray.astype(jnp.float32), sel,
                       preferred_element_type=jnp.float32)
        bitpos = jax.lax.broadcasted_iota(jnp.int32, (B, ENC), 1) % BITS
        bits = (gexp.astype(jnp.int32) >> bitpos) & 1
        enc_ref[...] = bits.astype(jnp.bfloat16)
        acc_ref[...] = jnp.zeros((B, C), jnp.float32)

    @pl.when(i < N1)
    def _layer1():
        w0 = w0_ref[...].astype(jnp.bfloat16)
        z = jax.lax.dot_general(enc_ref[...], w0, (((1,), (1,)), ((), ())),
                                preferred_element_type=jnp.float32)
        a0 = (z >= THR).astype(jnp.float32)
        a0_ref[:, pl.ds(i * BH1, BH1)] = a0.astype(jnp.bfloat16)
        c0 = outc_ref[0, pl.ds(i * BH1, BH1), :]
        acc_ref[...] += jnp.dot(a0, c0, preferred_element_type=jnp.float32)

    @pl.when(i >= N1)
    def _layer2():
        k = i - N1
        w1 = w1_ref[...].astype(jnp.bfloat16)
        z = jax.lax.dot_general(a0_ref[...], w1, (((1,), (1,)), ((), ())),
                                preferred_element_type=jnp.float32)
        a1 = (z >= THR).astype(jnp.float32)
        c1 = outc_ref[1, pl.ds(k * BH2, BH2), :]
        acc_ref[...] += jnp.dot(a1, c1, preferred_element_type=jnp.float32)

        @pl.when(k == N2 - 1)
        def _finish():
            out = acc_ref[...]
            outact_ref[...] = out
            preds_ref[0, :] = jnp.argmax(out, axis=1).astype(jnp.int32)


def kernel(trainOrTest, x, y, W0, W1, outC):
    preds2, outAct = pl.pallas_call(
        _fused_kernel,
        grid=(N1 + N2,),
        in_specs=[
            pl.BlockSpec((B, F), lambda i: (0, 0)),
            pl.BlockSpec((BH1, ENC), lambda i: (jnp.minimum(i, N1 - 1), 0)),
            pl.BlockSpec((BH2, H), lambda i: (jnp.maximum(i - N1, 0), 0)),
            pl.BlockSpec((2, H, C), lambda i: (0, 0, 0)),
        ],
        out_specs=[
            pl.BlockSpec((1, B), lambda i: (0, 0)),
            pl.BlockSpec((B, C), lambda i: (0, 0)),
        ],
        out_shape=[
            jax.ShapeDtypeStruct((1, B), jnp.int32),
            jax.ShapeDtypeStruct((B, C), jnp.float32),
        ],
        scratch_shapes=[
            pltpu.VMEM((B, ENC), jnp.bfloat16),
            pltpu.VMEM((B, H), jnp.bfloat16),
            pltpu.VMEM((B, C), jnp.float32),
        ],
        compiler_params=pltpu.CompilerParams(
            dimension_semantics=("arbitrary",),
        ),
    )(x, W0, W1, outC)
    return preds2[0], outAct
